# baseline (speedup 1.0000x reference)
import functools
import math

import jax
import jax.numpy as jnp
from jax import lax
from jax.experimental import pallas as pl
from jax.experimental.pallas import tpu as pltpu
from jax.experimental.pallas import tpu_sc as plsc

_N = 10000
_E = 320000
_D = 128
_NPAD = 10240
_NBLK = _NPAD // 128
_BI = 1024
_GRID = _NPAD // _BI

_NC = 2
_NS = 16
_NW = _NC * _NS
_CHUNK = _NPAD // _NW
_EFLAT = 2 * _E
_ECHUNK = _EFLAT // _NW


def _score_body(x_ref, w_ref, b_ref, s_ref, m_ref):
    i = pl.program_id(0)
    xb = x_ref[...]
    w = w_ref[...]
    wnorm = jnp.sqrt(jnp.sum(w * w))
    s = jnp.dot(xb, w.reshape(_D, 1), preferred_element_type=jnp.float32)
    s = (s + b_ref[0]) / wnorm
    s_ref[...] = jnp.transpose(s)
    ids = i * _BI + lax.broadcasted_iota(jnp.int32, (_BI, 1), 0)
    idf = lax.bitcast_convert_type(ids, jnp.float32)
    m_ref[...] = jnp.concatenate(
        [idf, jnp.tanh(s), jnp.zeros((_BI, _D - 2), jnp.float32)], axis=1)


def _scores(x_pad, W, b):
    return pl.pallas_call(
        _score_body,
        grid=(_GRID,),
        in_specs=[
            pl.BlockSpec((_BI, _D), lambda i: (i, 0)),
            pl.BlockSpec((1, _D), lambda i: (0, 0)),
            pl.BlockSpec(memory_space=pltpu.SMEM),
        ],
        out_specs=[
            pl.BlockSpec((1, _BI), lambda i: (0, i)),
            pl.BlockSpec((_BI, _D), lambda i: (i, 0)),
        ],
        out_shape=[
            jax.ShapeDtypeStruct((1, _NPAD), jnp.float32),
            jax.ShapeDtypeStruct((_NPAD, _D), jnp.float32),
        ],
    )(x_pad, W, b)


def _rank_body(si_ref, sj_ref, rank_ref, acc_c, acc_r):
    i = pl.program_id(0)
    j = pl.program_id(1)

    @pl.when((i == 0) & (j == 0))
    def _():
        acc_c[...] = jnp.zeros((1, _NPAD), jnp.float32)
        acc_r[...] = jnp.zeros((1, _NPAD), jnp.float32)

    @pl.when(j >= i)
    def _():
        ii = i * _BI + lax.broadcasted_iota(jnp.int32, (_BI, 1), 0)
        jj = j * _BI + lax.broadcasted_iota(jnp.int32, (1, _BI), 1)
        neg = jnp.float32(-jnp.inf)
        si = jnp.where(ii < _N, jnp.transpose(si_ref[...]), neg)
        sj = jnp.where(jj < _N, sj_ref[...], neg)
        gt = sj > si

        @pl.when(j == i)
        def _():
            cb = (gt | ((sj == si) & (jj < ii))).astype(jnp.float32)
            rs = jnp.sum(cb, axis=1, keepdims=True)
            acc_c[:, pl.ds(i * _BI, _BI)] += jnp.transpose(rs)

        @pl.when(j > i)
        def _():
            cb = gt.astype(jnp.float32)
            rs = jnp.sum(cb, axis=1, keepdims=True)
            acc_c[:, pl.ds(i * _BI, _BI)] += jnp.transpose(rs)
            cs = jnp.float32(_BI) - jnp.sum(cb, axis=0, keepdims=True)
            acc_r[:, pl.ds(j * _BI, _BI)] += cs

    @pl.when((i == _GRID - 1) & (j == _GRID - 1))
    def _():
        rank_ref[...] = jnp.clip(
            (acc_c[...] + acc_r[...]).astype(jnp.int32), 0, _N - 1)


def _ranks(scores_row):
    return pl.pallas_call(
        _rank_body,
        grid=(_GRID, _GRID),
        in_specs=[
            pl.BlockSpec((1, _BI), lambda i, j: (0, i)),
            pl.BlockSpec((1, _BI), lambda i, j: (0, j)),
        ],
        out_specs=pl.BlockSpec((1, _NPAD), lambda i, j: (0, 0)),
        out_shape=jax.ShapeDtypeStruct((1, _NPAD), jnp.int32),
        scratch_shapes=[pltpu.VMEM((1, _NPAD), jnp.float32),
                        pltpu.VMEM((1, _NPAD), jnp.float32)],
    )(scores_row, scores_row)


def _sc_body(x_hbm, rank_hbm, msrc_hbm, edge_hbm,
             xnew_hbm, meta_hbm, eout_hbm,
             rank_v, x_v, meta_v, eidx_v, eout_v, sh_rank,
             sem_stage, sem_e, sem_sc, sem_g):
    sid = lax.axis_index("s")
    wid = sid * _NC + lax.axis_index("c")
    base = jnp.minimum(wid * _CHUNK, _N - _CHUNK)
    ebase = wid * _ECHUNK

    cp_e = pltpu.make_async_copy(
        edge_hbm.at[pl.ds(ebase, _ECHUNK)], eidx_v, sem_e)
    cp_e.start()

    @pl.when(sid == 0)
    def _():
        pltpu.sync_copy(rank_hbm, sh_rank)
    plsc.subcore_barrier()
    cp_r = pltpu.make_async_copy(
        rank_hbm.at[pl.ds(base, _CHUNK)], rank_v, sem_stage)
    cp_r.start()
    cp_x = pltpu.make_async_copy(
        x_hbm.at[pl.ds(base, _CHUNK), :], x_v, sem_stage)
    cp_x.start()
    cp_m = pltpu.make_async_copy(
        msrc_hbm.at[pl.ds(base, _CHUNK), :], meta_v, sem_stage)
    cp_m.start()

    cp_r.wait()
    cp_x.wait()
    cp_m.wait()

    sc_x = pltpu.make_async_copy(x_v, xnew_hbm.at[rank_v], sem_sc)
    sc_x.start()
    sc_m = pltpu.make_async_copy(meta_v, meta_hbm.at[rank_v], sem_sc)
    sc_m.start()

    cp_e.wait()
    g = pltpu.make_async_copy(sh_rank.at[eidx_v], eout_v, sem_g)
    g.start()
    g.wait()
    pltpu.sync_copy(eout_v, eout_hbm.at[pl.ds(ebase, _ECHUNK)])

    sc_x.wait()
    sc_m.wait()


@functools.cache
def _make_sc_scatter():
    return functools.partial(
        pl.kernel,
        out_type=[
            jax.ShapeDtypeStruct((_N, _D), jnp.float32),
            jax.ShapeDtypeStruct((_N, _D), jnp.float32),
            jax.ShapeDtypeStruct((_EFLAT,), jnp.int32),
        ],
        mesh=plsc.VectorSubcoreMesh(
            core_axis_name="c", subcore_axis_name="s",
            num_cores=_NC, num_subcores=_NS),
        scratch_types=[
            pltpu.VMEM((_CHUNK,), jnp.int32),
            pltpu.VMEM((_CHUNK, _D), jnp.float32),
            pltpu.VMEM((_CHUNK, _D), jnp.float32),
            pltpu.VMEM((_ECHUNK,), jnp.int32),
            pltpu.VMEM((_ECHUNK,), jnp.int32),
            pltpu.VMEM_SHARED((_NPAD,), jnp.int32),
            pltpu.SemaphoreType.DMA,
            pltpu.SemaphoreType.DMA,
            pltpu.SemaphoreType.DMA,
            pltpu.SemaphoreType.DMA,
        ],
    )(_sc_body)


@jax.jit
def kernel(x, edge_index, epoch, W, b):
    scores_row, meta_src = _scores(x, W, b)

    rank_pad = _ranks(scores_row).reshape(_NPAD)

    x_new, meta, eout = _make_sc_scatter()(
        x, rank_pad, meta_src, edge_index.reshape(_EFLAT))

    scores = scores_row.reshape(_NPAD)[:_N]
    perm = lax.bitcast_convert_type(meta[:, 0], jnp.int32)
    scores_ranked = meta[:, 1:2]
    new_edge_index = eout.reshape(2, _E)
    return (x_new, new_edge_index, scores, perm, scores_ranked)


# speedup vs baseline: 1.0168x; 1.0168x over previous
import functools
import math

import jax
import jax.numpy as jnp
from jax import lax
from jax.experimental import pallas as pl
from jax.experimental.pallas import tpu as pltpu
from jax.experimental.pallas import tpu_sc as plsc

_N = 10000
_E = 320000
_D = 128
_NPAD = 10240
_NBLK = _NPAD // 128
_BI = 1024
_GRID = _NPAD // _BI

_NC = 2
_NS = 16
_NW = _NC * _NS
_CHUNK = _NPAD // _NW
_EFLAT = 2 * _E
_ECHUNK = _EFLAT // _NW


def _score_body(x_ref, w_ref, b_ref, s_ref, m_ref):
    i = pl.program_id(0)
    xb = x_ref[...]
    w = w_ref[...]
    wnorm = jnp.sqrt(jnp.sum(w * w))
    s = jnp.dot(xb, w.reshape(_D, 1), preferred_element_type=jnp.float32)
    s = (s + b_ref[0]) / wnorm
    s_ref[...] = s
    ids = i * _BI + lax.broadcasted_iota(jnp.int32, (_BI, 1), 0)
    idf = lax.bitcast_convert_type(ids, jnp.float32)
    m_ref[...] = jnp.concatenate(
        [idf, jnp.tanh(s), jnp.zeros((_BI, _D - 2), jnp.float32)], axis=1)


def _scores(x_pad, W, b):
    return pl.pallas_call(
        _score_body,
        grid=(_GRID,),
        in_specs=[
            pl.BlockSpec((_BI, _D), lambda i: (i, 0)),
            pl.BlockSpec((1, _D), lambda i: (0, 0)),
            pl.BlockSpec(memory_space=pltpu.SMEM),
        ],
        out_specs=[
            pl.BlockSpec((_BI, 1), lambda i: (i, 0)),
            pl.BlockSpec((_BI, _D), lambda i: (i, 0)),
        ],
        out_shape=[
            jax.ShapeDtypeStruct((_NPAD, 1), jnp.float32),
            jax.ShapeDtypeStruct((_NPAD, _D), jnp.float32),
        ],
    )(x_pad, W, b)


def _rank_body(si_ref, sj_ref, sum_c_ref, sum_r_ref, acc_r):
    i = pl.program_id(0)
    j = pl.program_id(1)

    @pl.when((i == 0) & (j == 0))
    def _():
        acc_r[...] = jnp.zeros((1, _NPAD), jnp.float32)

    @pl.when(j >= i)
    def _():
        ii = i * _BI + lax.broadcasted_iota(jnp.int32, (_BI, 1), 0)
        jj = j * _BI + lax.broadcasted_iota(jnp.int32, (1, _BI), 1)
        neg = jnp.float32(-jnp.inf)
        si = jnp.where(ii < _N, si_ref[...], neg)
        sj = jnp.where(jj < _N, sj_ref[...], neg)
        gt = sj > si

        @pl.when(j == i)
        def _():
            cb = (gt | ((sj == si) & (jj < ii))).astype(jnp.float32)
            sum_c_ref[...] = jnp.sum(cb, axis=1, keepdims=True)

        @pl.when(j > i)
        def _():
            cb = gt.astype(jnp.float32)
            sum_c_ref[...] += jnp.sum(cb, axis=1, keepdims=True)
            cs = jnp.float32(_BI) - jnp.sum(cb, axis=0, keepdims=True)
            acc_r[:, pl.ds(j * _BI, _BI)] += cs

    @pl.when((i == _GRID - 1) & (j == _GRID - 1))
    def _():
        sum_r_ref[...] = acc_r[...]


def _ranks(scores_col, scores_row):
    return pl.pallas_call(
        _rank_body,
        grid=(_GRID, _GRID),
        in_specs=[
            pl.BlockSpec((_BI, 1), lambda i, j: (i, 0)),
            pl.BlockSpec((1, _BI), lambda i, j: (0, j)),
        ],
        out_specs=[
            pl.BlockSpec((_BI, 1), lambda i, j: (i, 0)),
            pl.BlockSpec((1, _NPAD), lambda i, j: (0, 0)),
        ],
        out_shape=[
            jax.ShapeDtypeStruct((_NPAD, 1), jnp.float32),
            jax.ShapeDtypeStruct((1, _NPAD), jnp.float32),
        ],
        scratch_shapes=[pltpu.VMEM((1, _NPAD), jnp.float32)],
    )(scores_col, scores_row)


def _sc_body(x_hbm, rank_hbm, msrc_hbm, edge_hbm,
             xnew_hbm, meta_hbm, eout_hbm,
             rank_v, x_v, meta_v, eidx_v, eout_v, sh_rank,
             sem_stage, sem_e, sem_sc, sem_g):
    sid = lax.axis_index("s")
    wid = sid * _NC + lax.axis_index("c")
    base = jnp.minimum(wid * _CHUNK, _N - _CHUNK)
    ebase = wid * _ECHUNK

    cp_e = pltpu.make_async_copy(
        edge_hbm.at[pl.ds(ebase, _ECHUNK)], eidx_v, sem_e)
    cp_e.start()

    @pl.when(sid == 0)
    def _():
        pltpu.sync_copy(rank_hbm, sh_rank)
    plsc.subcore_barrier()
    cp_r = pltpu.make_async_copy(
        rank_hbm.at[pl.ds(base, _CHUNK)], rank_v, sem_stage)
    cp_r.start()
    cp_x = pltpu.make_async_copy(
        x_hbm.at[pl.ds(base, _CHUNK), :], x_v, sem_stage)
    cp_x.start()
    cp_m = pltpu.make_async_copy(
        msrc_hbm.at[pl.ds(base, _CHUNK), :], meta_v, sem_stage)
    cp_m.start()

    cp_r.wait()
    cp_x.wait()
    cp_m.wait()

    sc_x = pltpu.make_async_copy(x_v, xnew_hbm.at[rank_v], sem_sc)
    sc_x.start()
    sc_m = pltpu.make_async_copy(meta_v, meta_hbm.at[rank_v], sem_sc)
    sc_m.start()

    cp_e.wait()
    g = pltpu.make_async_copy(sh_rank.at[eidx_v], eout_v, sem_g)
    g.start()
    g.wait()
    pltpu.sync_copy(eout_v, eout_hbm.at[pl.ds(ebase, _ECHUNK)])

    sc_x.wait()
    sc_m.wait()


@functools.cache
def _make_sc_scatter():
    return functools.partial(
        pl.kernel,
        out_type=[
            jax.ShapeDtypeStruct((_N, _D), jnp.float32),
            jax.ShapeDtypeStruct((_N, _D), jnp.float32),
            jax.ShapeDtypeStruct((_EFLAT,), jnp.int32),
        ],
        mesh=plsc.VectorSubcoreMesh(
            core_axis_name="c", subcore_axis_name="s",
            num_cores=_NC, num_subcores=_NS),
        scratch_types=[
            pltpu.VMEM((_CHUNK,), jnp.int32),
            pltpu.VMEM((_CHUNK, _D), jnp.float32),
            pltpu.VMEM((_CHUNK, _D), jnp.float32),
            pltpu.VMEM((_ECHUNK,), jnp.int32),
            pltpu.VMEM((_ECHUNK,), jnp.int32),
            pltpu.VMEM_SHARED((_NPAD,), jnp.int32),
            pltpu.SemaphoreType.DMA,
            pltpu.SemaphoreType.DMA,
            pltpu.SemaphoreType.DMA,
            pltpu.SemaphoreType.DMA,
        ],
    )(_sc_body)


@jax.jit
def kernel(x, edge_index, epoch, W, b):
    scores_col, meta_src = _scores(x, W, b)

    sum_c, sum_r = _ranks(scores_col, scores_col.reshape(1, _NPAD))
    rank_pad = jnp.clip(
        (sum_c.reshape(_NPAD) + sum_r.reshape(_NPAD)).astype(jnp.int32),
        0, _N - 1)

    x_new, meta, eout = _make_sc_scatter()(
        x, rank_pad, meta_src, edge_index.reshape(_EFLAT))

    scores = scores_col.reshape(_NPAD)[:_N]
    perm = lax.bitcast_convert_type(meta[:, 0], jnp.int32)
    scores_ranked = meta[:, 1:2]
    new_edge_index = eout.reshape(2, _E)
    return (x_new, new_edge_index, scores, perm, scores_ranked)


# speedup vs baseline: 1.0898x; 1.0718x over previous
import functools
import math

import jax
import jax.numpy as jnp
from jax import lax
from jax.experimental import pallas as pl
from jax.experimental.pallas import tpu as pltpu
from jax.experimental.pallas import tpu_sc as plsc

_N = 10000
_E = 320000
_D = 128
_NPAD = 10240
_NBLK = _NPAD // 128
_BI = 1024
_GRID = _NPAD // _BI

_NC = 2
_NS = 16
_NW = _NC * _NS
_CHUNK = _NPAD // _NW
_EFLAT = 2 * _E
_ECHUNK = 20480
_ESTRIDE = 19968


def _score_body(x_ref, w_ref, b_ref, s_ref, m_ref):
    i = pl.program_id(0)
    xb = x_ref[...]
    w = w_ref[...]
    wnorm = jnp.sqrt(jnp.sum(w * w))
    s = jnp.dot(xb, w.reshape(_D, 1), preferred_element_type=jnp.float32)
    s = (s + b_ref[0]) / wnorm
    s_ref[...] = s
    ids = i * _BI + lax.broadcasted_iota(jnp.int32, (_BI, 1), 0)
    idf = lax.bitcast_convert_type(ids, jnp.float32)
    m_ref[...] = jnp.concatenate(
        [idf, jnp.tanh(s), jnp.zeros((_BI, _D - 2), jnp.float32)], axis=1)


def _scores(x_pad, W, b):
    return pl.pallas_call(
        _score_body,
        grid=(_GRID,),
        in_specs=[
            pl.BlockSpec((_BI, _D), lambda i: (i, 0)),
            pl.BlockSpec((1, _D), lambda i: (0, 0)),
            pl.BlockSpec(memory_space=pltpu.SMEM),
        ],
        out_specs=[
            pl.BlockSpec((_BI, 1), lambda i: (i, 0)),
            pl.BlockSpec((_BI, _D), lambda i: (i, 0)),
        ],
        out_shape=[
            jax.ShapeDtypeStruct((_NPAD, 1), jnp.float32),
            jax.ShapeDtypeStruct((_NPAD, _D), jnp.float32),
        ],
    )(x_pad, W, b)


def _rank_body(si_ref, sj_ref, sum_c_ref, sum_r_ref, acc_r):
    i = pl.program_id(0)
    j = pl.program_id(1)

    @pl.when((i == 0) & (j == 0))
    def _():
        acc_r[...] = jnp.zeros((1, _NPAD), jnp.float32)

    @pl.when(j >= i)
    def _():
        ii = i * _BI + lax.broadcasted_iota(jnp.int32, (_BI, 1), 0)
        jj = j * _BI + lax.broadcasted_iota(jnp.int32, (1, _BI), 1)
        neg = jnp.float32(-jnp.inf)
        si = jnp.where(ii < _N, si_ref[...], neg)
        sj = jnp.where(jj < _N, sj_ref[...], neg)
        gt = sj > si

        @pl.when(j == i)
        def _():
            cb = (gt | ((sj == si) & (jj < ii))).astype(jnp.float32)
            sum_c_ref[...] = jnp.sum(cb, axis=1, keepdims=True)

        @pl.when(j > i)
        def _():
            cb = gt.astype(jnp.float32)
            sum_c_ref[...] += jnp.sum(cb, axis=1, keepdims=True)
            cs = jnp.float32(_BI) - jnp.sum(cb, axis=0, keepdims=True)
            acc_r[:, pl.ds(j * _BI, _BI)] += cs

    @pl.when((i == _GRID - 1) & (j == _GRID - 1))
    def _():
        sum_r_ref[...] = acc_r[...]


def _ranks(scores_col, scores_row):
    return pl.pallas_call(
        _rank_body,
        grid=(_GRID, _GRID),
        in_specs=[
            pl.BlockSpec((_BI, 1), lambda i, j: (i, 0)),
            pl.BlockSpec((1, _BI), lambda i, j: (0, j)),
        ],
        out_specs=[
            pl.BlockSpec((_BI, 1), lambda i, j: (i, 0)),
            pl.BlockSpec((1, _NPAD), lambda i, j: (0, 0)),
        ],
        out_shape=[
            jax.ShapeDtypeStruct((_NPAD, 1), jnp.float32),
            jax.ShapeDtypeStruct((1, _NPAD), jnp.float32),
        ],
        scratch_shapes=[pltpu.VMEM((1, _NPAD), jnp.float32)],
    )(scores_col, scores_row)


def _sc_body(x_hbm, rank_hbm, msrc_hbm, edge_hbm,
             xnew_hbm, meta_hbm, eout_hbm,
             rank_v, x_v, meta_v, eidx_v, eout_v, sh_rank,
             sem_stage, sem_e, sem_sc, sem_g):
    sid = lax.axis_index("s")
    wid = sid * _NC + lax.axis_index("c")
    base = jnp.minimum(wid * _CHUNK, _N - _CHUNK)
    erow = wid // 16
    ecol = (wid % 16) * _ESTRIDE

    cp_e = pltpu.make_async_copy(
        edge_hbm.at[erow, pl.ds(ecol, _ECHUNK)], eidx_v, sem_e)
    cp_e.start()

    @pl.when(sid == 0)
    def _():
        pltpu.sync_copy(rank_hbm, sh_rank)
    plsc.subcore_barrier()
    cp_r = pltpu.make_async_copy(
        rank_hbm.at[pl.ds(base, _CHUNK)], rank_v, sem_stage)
    cp_r.start()
    cp_x = pltpu.make_async_copy(
        x_hbm.at[pl.ds(base, _CHUNK), :], x_v, sem_stage)
    cp_x.start()
    cp_m = pltpu.make_async_copy(
        msrc_hbm.at[pl.ds(base, _CHUNK), :], meta_v, sem_stage)
    cp_m.start()

    cp_r.wait()
    cp_x.wait()
    cp_m.wait()

    sc_x = pltpu.make_async_copy(x_v, xnew_hbm.at[rank_v], sem_sc)
    sc_x.start()
    sc_m = pltpu.make_async_copy(meta_v, meta_hbm.at[rank_v], sem_sc)
    sc_m.start()

    cp_e.wait()
    g = pltpu.make_async_copy(sh_rank.at[eidx_v], eout_v, sem_g)
    g.start()
    g.wait()
    pltpu.sync_copy(eout_v, eout_hbm.at[erow, pl.ds(ecol, _ECHUNK)])

    sc_x.wait()
    sc_m.wait()


@functools.cache
def _make_sc_scatter():
    return functools.partial(
        pl.kernel,
        out_type=[
            jax.ShapeDtypeStruct((_N, _D), jnp.float32),
            jax.ShapeDtypeStruct((_N, _D), jnp.float32),
            jax.ShapeDtypeStruct((2, _E), jnp.int32),
        ],
        mesh=plsc.VectorSubcoreMesh(
            core_axis_name="c", subcore_axis_name="s",
            num_cores=_NC, num_subcores=_NS),
        scratch_types=[
            pltpu.VMEM((_CHUNK,), jnp.int32),
            pltpu.VMEM((_CHUNK, _D), jnp.float32),
            pltpu.VMEM((_CHUNK, _D), jnp.float32),
            pltpu.VMEM((_ECHUNK,), jnp.int32),
            pltpu.VMEM((_ECHUNK,), jnp.int32),
            pltpu.VMEM_SHARED((_NPAD,), jnp.int32),
            pltpu.SemaphoreType.DMA,
            pltpu.SemaphoreType.DMA,
            pltpu.SemaphoreType.DMA,
            pltpu.SemaphoreType.DMA,
        ],
    )(_sc_body)


@jax.jit
def kernel(x, edge_index, epoch, W, b):
    scores_col, meta_src = _scores(x, W, b)

    sum_c, sum_r = _ranks(scores_col, scores_col.reshape(1, _NPAD))
    rank_pad = jnp.clip(
        (sum_c.reshape(_NPAD) + sum_r.reshape(_NPAD)).astype(jnp.int32),
        0, _N - 1)

    x_new, meta, new_edge_index = _make_sc_scatter()(
        x, rank_pad, meta_src, edge_index)

    scores = scores_col.reshape(_NPAD)[:_N]
    perm = lax.bitcast_convert_type(meta[:, 0], jnp.int32)
    scores_ranked = meta[:, 1:2]
    return (x_new, new_edge_index, scores, perm, scores_ranked)
